# Initial kernel scaffold; baseline (speedup 1.0000x reference)
import jax
import jax.numpy as jnp
from jax.experimental import pallas as pl


def kernel(x, edge_index, batch, edge_weight, Wf, bf, W1, b1, W2, b2, W3, b3, Wl, bl, Wc, bc):
    raise NotImplementedError("write your pallas kernel here")


# speedup vs baseline: 1.8065x; 1.8065x over previous
import jax
import jax.numpy as jnp
from jax.experimental import pallas as pl


def _bn(x):
    mu = x.mean(axis=0)
    var = x.var(axis=0)
    return (x - mu) / jnp.sqrt(var + 1e-5) + 1e-4


def kernel(x, edge_index, batch, edge_weight, Wf, bf, W1, b1, W2, b2, W3, b3, Wl, bl, Wc, bc):
    n = x.shape[0]
    src, dst = edge_index[0], edge_index[1]
    deg = jnp.zeros((n,), x.dtype).at[dst].add(edge_weight) + 1.0
    dinv = 1.0 / jnp.sqrt(deg)

    def gcn(h, W, b):
        hp = (_bn(h) @ W) * dinv[:, None]
        agg = jnp.zeros_like(hp).at[dst].add(hp[src] * edge_weight[:, None])
        return jax.nn.relu((agg + hp) * dinv[:, None] + b)

    x0 = gcn(x, Wf, bf)
    convs = [(W1, b1), (W2, b2), (W3, b3)]
    h = x0
    for (W, b) in convs:
        h = gcn(h, W, b)
    h1 = h
    h = x0
    for (W, b) in convs:
        h = gcn(h, W, b)
        h = gcn(h, W, b)
    h2 = h
    h4 = x0
    for _ in convs:
        h4 = jax.nn.relu(_bn(h4))

    outs = []
    for h in (h1, h2, h4):
        g = jax.ops.segment_sum(h, batch, num_segments=128)
        g = jax.nn.relu(_bn(g) @ Wl + bl)
        g = _bn(g) @ Wc + bc
        outs.append(jax.nn.log_softmax(g, axis=-1))
    return (outs[0], outs[1], outs[0], outs[2])


# speedup vs baseline: 5.9243x; 3.2794x over previous
import functools

import jax
import jax.numpy as jnp
from jax import lax
from jax.experimental import pallas as pl
from jax.experimental.pallas import tpu as pltpu
from jax.experimental.pallas import tpu_sc as plsc

N = 10000
E = 320000
H = 128
G = 128

_NC = 2
_NS = 16
_NW = _NC * _NS
_EPW = E // _NW
_RPT = N // _NS

_DEG_CH = 1000
_ACH = 80

_sc_mesh = plsc.VectorSubcoreMesh(core_axis_name="c", subcore_axis_name="s")


def _deg_body(dst_hbm, ew_hbm, outa_hbm, outb_hbm,
              acc1d, didx, wv, zb1):
    c = lax.axis_index("c")
    s = lax.axis_index("s")
    wid = s * _NC + c
    rbase = pl.multiple_of(s * 640, 8)

    def zero(i, carry):
        zb1[pl.ds(i * 16, 16)] = jnp.zeros((16,), jnp.float32)
        return carry

    lax.fori_loop(0, 40, zero, 0)

    @pl.when(s < 15)
    def _():
        pltpu.sync_copy(zb1, acc1d.at[pl.ds(rbase, 640)])

    @pl.when(s == 15)
    def _():
        pltpu.sync_copy(zb1.at[pl.ds(0, 400)], acc1d.at[pl.ds(rbase, 400)])

    plsc.subcore_barrier()

    ebase = wid * _EPW

    def chunk(ci, carry):
        b = ebase + ci * _ACH
        pltpu.sync_copy(dst_hbm.at[pl.ds(b, _ACH)], didx)
        pltpu.sync_copy(ew_hbm.at[pl.ds(b, _ACH)], wv)
        pltpu.sync_copy(wv, acc1d.at[didx], add=True)
        return carry

    lax.fori_loop(0, _EPW // _ACH, chunk, 0)
    plsc.subcore_barrier()

    for cc, out_ref in ((0, outa_hbm), (1, outb_hbm)):
        @pl.when((c == cc) & (s < 15))
        def _(out_ref=out_ref):
            pltpu.sync_copy(acc1d.at[pl.ds(rbase, 640)], zb1)
            pltpu.sync_copy(zb1, out_ref.at[pl.ds(rbase, 640)])

        @pl.when((c == cc) & (s == 15))
        def _(out_ref=out_ref):
            pltpu.sync_copy(acc1d.at[pl.ds(rbase, 400)], zb1.at[pl.ds(0, 400)])
            pltpu.sync_copy(zb1.at[pl.ds(0, 400)], out_ref.at[pl.ds(rbase, 400)])


@jax.jit
def _sc_deg(dst, ew):
    k = pl.kernel(
        _deg_body,
        mesh=_sc_mesh,
        out_type=[
            jax.ShapeDtypeStruct((N,), jnp.float32),
            jax.ShapeDtypeStruct((N,), jnp.float32),
        ],
        scratch_types=[
            pltpu.VMEM_SHARED((N,), jnp.float32),
            pltpu.VMEM((_ACH,), jnp.int32),
            pltpu.VMEM((_ACH,), jnp.float32),
            pltpu.VMEM((640,), jnp.float32),
        ],
    )
    return k(dst, ew)


def _agg_body(hp_hbm, src_hbm, dst_hbm, ew_hbm, outa_hbm, outb_hbm,
              acc_sh, rows, sidx, didx, wv, zbuf, gsem):
    c = lax.axis_index("c")
    s = lax.axis_index("s")
    wid = s * _NC + c
    rbase = pl.multiple_of(s * 640, 8)

    def zero(i, carry):
        for j in range(8):
            zbuf[i, pl.ds(16 * j, 16)] = jnp.zeros((16,), jnp.float32)
        return carry

    lax.fori_loop(0, 80, zero, 0)

    @pl.when(s < 15)
    def _():
        for k in range(8):
            pltpu.sync_copy(zbuf, acc_sh.at[pl.ds(rbase + 80 * k, 80)])

    @pl.when(s == 15)
    def _():
        for k in range(5):
            pltpu.sync_copy(zbuf, acc_sh.at[pl.ds(rbase + 80 * k, 80)])

    plsc.subcore_barrier()

    ebase = wid * _EPW

    def chunk(ci, carry):
        b = ebase + ci * _ACH
        pltpu.sync_copy(src_hbm.at[pl.ds(b, _ACH)], sidx)
        pltpu.sync_copy(dst_hbm.at[pl.ds(b, _ACH)], didx)
        pltpu.sync_copy(ew_hbm.at[pl.ds(b, _ACH)], wv)
        pltpu.async_copy(hp_hbm.at[sidx], rows, gsem).wait()

        def sgrp(q, carry2):
            w16 = wv[pl.ds(q * 16, 16)]
            for l in range(16):
                r = q * 16 + l
                wb = lax.broadcast_in_dim(w16[l], (16,), ())
                for j in range(8):
                    rows[r, pl.ds(16 * j, 16)] = rows[r, pl.ds(16 * j, 16)] * wb
            return carry2

        lax.fori_loop(0, _ACH // 16, sgrp, 0)
        pltpu.sync_copy(rows, acc_sh.at[didx], add=True)
        return carry

    lax.fori_loop(0, _EPW // _ACH, chunk, 0)
    plsc.subcore_barrier()

    for cc, out_ref in ((0, outa_hbm), (1, outb_hbm)):
        @pl.when((c == cc) & (s < 15))
        def _(out_ref=out_ref):
            pltpu.sync_copy(acc_sh.at[pl.ds(rbase, 640)],
                            out_ref.at[pl.ds(rbase, 640)])

        @pl.when((c == cc) & (s == 15))
        def _(out_ref=out_ref):
            pltpu.sync_copy(acc_sh.at[pl.ds(rbase, 400)],
                            out_ref.at[pl.ds(rbase, 400)])


@jax.jit
def _sc_agg(hp, src, dst, ew):
    k = pl.kernel(
        _agg_body,
        mesh=_sc_mesh,
        out_type=[
            jax.ShapeDtypeStruct((N, H), jnp.float32),
            jax.ShapeDtypeStruct((N, H), jnp.float32),
        ],
        scratch_types=[
            pltpu.VMEM_SHARED((N, H), jnp.float32),
            pltpu.VMEM((_ACH, H), jnp.float32),
            pltpu.VMEM((_ACH,), jnp.int32),
            pltpu.VMEM((_ACH,), jnp.int32),
            pltpu.VMEM((_ACH,), jnp.float32),
            pltpu.VMEM((80, H), jnp.float32),
            pltpu.SemaphoreType.DMA,
        ],
    )
    return k(hp, src, dst, ew)


def _bn_in(x):
    mu = jnp.mean(x, axis=0, keepdims=True)
    xc = x - mu
    var = jnp.mean(xc * xc, axis=0, keepdims=True)
    return xc * lax.rsqrt(var + 1e-5) + 1e-4


def _pre_body(h_ref, w_ref, dinv_ref, o_ref):
    xn = _bn_in(h_ref[...])
    o_ref[...] = jnp.dot(xn, w_ref[...],
                         preferred_element_type=jnp.float32) * dinv_ref[...]


def _tc_pre(h, W, dinv):
    return pl.pallas_call(
        _pre_body,
        out_shape=jax.ShapeDtypeStruct((N, H), jnp.float32),
    )(h, W, dinv)


def _post_body(aa_ref, ab_ref, hp_ref, dinv_ref, b_ref, o_ref):
    t = (aa_ref[...] + ab_ref[...] + hp_ref[...]) * dinv_ref[...] + b_ref[...]
    o_ref[...] = jnp.maximum(t, 0.0)


def _tc_post(aa, ab, hp, dinv, b):
    return pl.pallas_call(
        _post_body,
        out_shape=jax.ShapeDtypeStruct((N, H), jnp.float32),
    )(aa, ab, hp, dinv, b)


def _bn3_body(h_ref, o_ref):
    h = h_ref[...]
    for _ in range(3):
        h = jnp.maximum(_bn_in(h), 0.0)
    o_ref[...] = h


def _tc_bn3(h):
    return pl.pallas_call(
        _bn3_body,
        out_shape=jax.ShapeDtypeStruct((N, H), jnp.float32),
    )(h)


def _head_body(h1_ref, h2_ref, h4_ref, batch_ref, wl_ref, bl_ref,
               wc_ref, bc_ref, o1_ref, o2_ref, o4_ref):
    iota_g = lax.broadcasted_iota(jnp.int32, (G, N), 0)
    m = (batch_ref[...] == iota_g).astype(jnp.float32)
    for h_ref, o_ref in ((h1_ref, o1_ref), (h2_ref, o2_ref), (h4_ref, o4_ref)):
        g = jnp.dot(m, h_ref[...], preferred_element_type=jnp.float32)
        g = _bn_in(g)
        g = jnp.maximum(
            jnp.dot(g, wl_ref[...], preferred_element_type=jnp.float32)
            + bl_ref[...], 0.0)
        g = _bn_in(g)
        g = jnp.dot(g, wc_ref[...], preferred_element_type=jnp.float32) \
            + bc_ref[...]
        mx = jnp.max(g, axis=1, keepdims=True)
        lse = jnp.log(jnp.sum(jnp.exp(g - mx), axis=1, keepdims=True)) + mx
        o_ref[...] = g - lse


def _tc_head(h1, h2, h4, batch2d, Wl, bl, Wc, bc):
    return pl.pallas_call(
        _head_body,
        out_shape=[
            jax.ShapeDtypeStruct((G, 10), jnp.float32),
            jax.ShapeDtypeStruct((G, 10), jnp.float32),
            jax.ShapeDtypeStruct((G, 10), jnp.float32),
        ],
    )(h1, h2, h4, batch2d, Wl, bl, Wc, bc)


def kernel(x, edge_index, batch, edge_weight, Wf, bf, W1, b1, W2, b2, W3, b3,
           Wl, bl, Wc, bc):
    src = edge_index[0]
    dst = edge_index[1]

    dega, degb = _sc_deg(dst, edge_weight)
    dinv = (1.0 / jnp.sqrt(dega + degb + 1.0)).reshape(N, 1)

    def conv(h, W, b):
        hp = _tc_pre(h, W, dinv)
        aa, ab = _sc_agg(hp, src, dst, edge_weight)
        return _tc_post(aa, ab, hp, dinv, b.reshape(1, H))

    x0 = conv(x, Wf, bf)
    convs = [(W1, b1), (W2, b2), (W3, b3)]

    h = x0
    for (W, b) in convs:
        h = conv(h, W, b)
    h1 = h

    h = x0
    for (W, b) in convs:
        h = conv(h, W, b)
        h = conv(h, W, b)
    h2 = h

    h4 = _tc_bn3(x0)

    o1, o2, o4 = _tc_head(h1, h2, h4, batch.reshape(1, N), Wl,
                          bl.reshape(1, H), Wc, bc.reshape(1, 10))
    return (o1, o2, o1, o4)


# speedup vs baseline: 16.5428x; 2.7924x over previous
import functools

import jax
import jax.numpy as jnp
from jax import lax
from jax.experimental import pallas as pl
from jax.experimental.pallas import tpu as pltpu
from jax.experimental.pallas import tpu_sc as plsc

N = 10000
E = 320000
H = 128
G = 128

_NC = 2
_NS = 16
_NW = _NC * _NS
_EPW = E // _NW
_RPT = N // _NS

_ACH = 128
_NCH = 80
_EPT = _NCH * _ACH
_EPAD = _NW * _EPT

_sc_mesh = plsc.VectorSubcoreMesh(core_axis_name="c", subcore_axis_name="s")


def _deg_body(dst_hbm, ew_hbm, outa_hbm, outb_hbm,
              acc1d, didx2d, wv2d, zb1):
    c = lax.axis_index("c")
    s = lax.axis_index("s")
    wid = s * _NC + c
    rbase = pl.multiple_of(s * 640, 8)

    def zero(i, carry):
        zb1[pl.ds(i * 16, 16)] = jnp.zeros((16,), jnp.float32)
        return carry

    lax.fori_loop(0, 40, zero, 0)

    @pl.when(s < 15)
    def _():
        pltpu.sync_copy(zb1, acc1d.at[pl.ds(rbase, 640)])

    @pl.when(s == 15)
    def _():
        pltpu.sync_copy(zb1.at[pl.ds(0, 400)], acc1d.at[pl.ds(rbase, 400)])

    pltpu.sync_copy(dst_hbm.at[wid], didx2d)
    pltpu.sync_copy(ew_hbm.at[wid], wv2d)
    plsc.subcore_barrier()

    def chunk(q, carry):
        pltpu.sync_copy(wv2d.at[q], acc1d.at[didx2d.at[q]], add=True)
        return carry

    lax.fori_loop(0, _NCH, chunk, 0)
    plsc.subcore_barrier()

    for cc, out_ref in ((0, outa_hbm), (1, outb_hbm)):
        @pl.when((c == cc) & (s < 15))
        def _(out_ref=out_ref):
            pltpu.sync_copy(acc1d.at[pl.ds(rbase, 640)], zb1)
            pltpu.sync_copy(zb1, out_ref.at[pl.ds(rbase, 640)])

        @pl.when((c == cc) & (s == 15))
        def _(out_ref=out_ref):
            pltpu.sync_copy(acc1d.at[pl.ds(rbase, 400)], zb1.at[pl.ds(0, 400)])
            pltpu.sync_copy(zb1.at[pl.ds(0, 400)], out_ref.at[pl.ds(rbase, 400)])


@jax.jit
def _sc_deg(dst3, ew3):
    k = pl.kernel(
        _deg_body,
        mesh=_sc_mesh,
        out_type=[
            jax.ShapeDtypeStruct((N,), jnp.float32),
            jax.ShapeDtypeStruct((N,), jnp.float32),
        ],
        scratch_types=[
            pltpu.VMEM_SHARED((N,), jnp.float32),
            pltpu.VMEM((_NCH, _ACH), jnp.int32),
            pltpu.VMEM((_NCH, _ACH), jnp.float32),
            pltpu.VMEM((640,), jnp.float32),
        ],
    )
    return k(dst3, ew3)


def _agg_body(hp_hbm, src_hbm, dst_hbm, ew_hbm, outa_hbm, outb_hbm,
              acc_sh, rows3, sidx2d, didx2d, wv2d, gsem0, gsem1):
    c = lax.axis_index("c")
    s = lax.axis_index("s")
    wid = s * _NC + c
    rbase = pl.multiple_of(s * 640, 8)
    gsems = (gsem0, gsem1)

    def zero(i, carry):
        for j in range(8):
            rows3[0, i, pl.ds(16 * j, 16)] = jnp.zeros((16,), jnp.float32)
        return carry

    lax.fori_loop(0, _ACH, zero, 0)
    z128 = rows3.at[0]

    @pl.when(s < 15)
    def _():
        for k in range(5):
            pltpu.sync_copy(z128, acc_sh.at[pl.ds(rbase + 128 * k, 128)])

    @pl.when(s == 15)
    def _():
        for k in range(3):
            pltpu.sync_copy(z128, acc_sh.at[pl.ds(rbase + 128 * k, 128)])
        pltpu.sync_copy(z128.at[pl.ds(0, 16)],
                        acc_sh.at[pl.ds(rbase + 384, 16)])

    plsc.subcore_barrier()

    for half in range(2):
        hbase = half * (_NCH // 2)
        pltpu.sync_copy(src_hbm.at[wid, pl.ds(hbase, _NCH // 2)], sidx2d)
        pltpu.sync_copy(dst_hbm.at[wid, pl.ds(hbase, _NCH // 2)], didx2d)
        pltpu.sync_copy(ew_hbm.at[wid, pl.ds(hbase, _NCH // 2)], wv2d)
        pltpu.async_copy(hp_hbm.at[sidx2d.at[0]], rows3.at[0], gsem0)

        def pair(i, carry):
            for b in range(2):
                q = i * 2 + b

                @pl.when(q + 1 < _NCH // 2)
                def _():
                    pltpu.async_copy(hp_hbm.at[sidx2d.at[q + 1]],
                                     rows3.at[1 - b], gsems[1 - b])

                pltpu.make_async_copy(hp_hbm.at[sidx2d.at[q]],
                                      rows3.at[b], gsems[b]).wait()

                def sgrp(g, carry2):
                    w16 = wv2d[q, pl.ds(g * 16, 16)]
                    for l in range(16):
                        r = g * 16 + l
                        wb = lax.broadcast_in_dim(w16[l], (16,), ())
                        for j in range(8):
                            rows3[b, r, pl.ds(16 * j, 16)] = (
                                rows3[b, r, pl.ds(16 * j, 16)] * wb)
                    return carry2

                lax.fori_loop(0, _ACH // 16, sgrp, 0)
                pltpu.sync_copy(rows3.at[b], acc_sh.at[didx2d.at[q]], add=True)
            return carry

        lax.fori_loop(0, _NCH // 4, pair, 0)

    plsc.subcore_barrier()

    for cc, out_ref in ((0, outa_hbm), (1, outb_hbm)):
        @pl.when((c == cc) & (s < 15))
        def _(out_ref=out_ref):
            pltpu.sync_copy(acc_sh.at[pl.ds(rbase, 640)],
                            out_ref.at[pl.ds(rbase, 640)])

        @pl.when((c == cc) & (s == 15))
        def _(out_ref=out_ref):
            pltpu.sync_copy(acc_sh.at[pl.ds(rbase, 400)],
                            out_ref.at[pl.ds(rbase, 400)])


@jax.jit
def _sc_agg(hp, src, dst, ew):
    k = pl.kernel(
        _agg_body,
        mesh=_sc_mesh,
        out_type=[
            jax.ShapeDtypeStruct((N, H), jnp.float32),
            jax.ShapeDtypeStruct((N, H), jnp.float32),
        ],
        scratch_types=[
            pltpu.VMEM_SHARED((N, H), jnp.float32),
            pltpu.VMEM((2, _ACH, H), jnp.float32),
            pltpu.VMEM((_NCH // 2, _ACH), jnp.int32),
            pltpu.VMEM((_NCH // 2, _ACH), jnp.int32),
            pltpu.VMEM((_NCH // 2, _ACH), jnp.float32),
            pltpu.SemaphoreType.DMA,
            pltpu.SemaphoreType.DMA,
        ],
    )
    return k(hp, src, dst, ew)


def _bn_in(x):
    mu = jnp.mean(x, axis=0, keepdims=True)
    xc = x - mu
    var = jnp.mean(xc * xc, axis=0, keepdims=True)
    return xc * lax.rsqrt(var + 1e-5) + 1e-4


def _pre_body(h_ref, w_ref, dinv_ref, o_ref):
    xn = _bn_in(h_ref[...])
    o_ref[...] = jnp.dot(xn, w_ref[...],
                         preferred_element_type=jnp.float32) * dinv_ref[...]


def _tc_pre(h, W, dinv):
    return pl.pallas_call(
        _pre_body,
        out_shape=jax.ShapeDtypeStruct((N, H), jnp.float32),
    )(h, W, dinv)


def _post_body(aa_ref, ab_ref, hp_ref, dinv_ref, b_ref, o_ref):
    t = (aa_ref[...] + ab_ref[...] + hp_ref[...]) * dinv_ref[...] + b_ref[...]
    o_ref[...] = jnp.maximum(t, 0.0)


def _tc_post(aa, ab, hp, dinv, b):
    return pl.pallas_call(
        _post_body,
        out_shape=jax.ShapeDtypeStruct((N, H), jnp.float32),
    )(aa, ab, hp, dinv, b)


def _bn3_body(h_ref, o_ref):
    h = h_ref[...]
    for _ in range(3):
        h = jnp.maximum(_bn_in(h), 0.0)
    o_ref[...] = h


def _tc_bn3(h):
    return pl.pallas_call(
        _bn3_body,
        out_shape=jax.ShapeDtypeStruct((N, H), jnp.float32),
    )(h)


def _head_body(h1_ref, h2_ref, h4_ref, batch_ref, wl_ref, bl_ref,
               wc_ref, bc_ref, o1_ref, o2_ref, o4_ref):
    iota_g = lax.broadcasted_iota(jnp.int32, (G, N), 0)
    m = (batch_ref[...] == iota_g).astype(jnp.float32)
    for h_ref, o_ref in ((h1_ref, o1_ref), (h2_ref, o2_ref), (h4_ref, o4_ref)):
        g = jnp.dot(m, h_ref[...], preferred_element_type=jnp.float32)
        g = _bn_in(g)
        g = jnp.maximum(
            jnp.dot(g, wl_ref[...], preferred_element_type=jnp.float32)
            + bl_ref[...], 0.0)
        g = _bn_in(g)
        g = jnp.dot(g, wc_ref[...], preferred_element_type=jnp.float32) \
            + bc_ref[...]
        mx = jnp.max(g, axis=1, keepdims=True)
        lse = jnp.log(jnp.sum(jnp.exp(g - mx), axis=1, keepdims=True)) + mx
        o_ref[...] = g - lse


def _tc_head(h1, h2, h4, batch2d, Wl, bl, Wc, bc):
    return pl.pallas_call(
        _head_body,
        out_shape=[
            jax.ShapeDtypeStruct((G, 10), jnp.float32),
            jax.ShapeDtypeStruct((G, 10), jnp.float32),
            jax.ShapeDtypeStruct((G, 10), jnp.float32),
        ],
    )(h1, h2, h4, batch2d, Wl, bl, Wc, bc)


def kernel(x, edge_index, batch, edge_weight, Wf, bf, W1, b1, W2, b2, W3, b3,
           Wl, bl, Wc, bc):
    pad = _EPAD - E
    fill = (jnp.arange(pad, dtype=jnp.int32) * 797) % N
    src3 = jnp.concatenate([edge_index[0], fill]).reshape(_NW, _NCH, _ACH)
    dst3 = jnp.concatenate([edge_index[1], fill]).reshape(_NW, _NCH, _ACH)
    ew3 = jnp.concatenate(
        [edge_weight, jnp.zeros((pad,), jnp.float32)]).reshape(_NW, _NCH, _ACH)

    dega, degb = _sc_deg(dst3, ew3)
    dinv = (1.0 / jnp.sqrt(dega + degb + 1.0)).reshape(N, 1)

    def conv(h, W, b):
        hp = _tc_pre(h, W, dinv)
        aa, ab = _sc_agg(hp, src3, dst3, ew3)
        return _tc_post(aa, ab, hp, dinv, b.reshape(1, H))

    x0 = conv(x, Wf, bf)
    convs = [(W1, b1), (W2, b2), (W3, b3)]

    h = x0
    for (W, b) in convs:
        h = conv(h, W, b)
    h1 = h

    h = x0
    for (W, b) in convs:
        h = conv(h, W, b)
        h = conv(h, W, b)
    h2 = h

    h4 = _tc_bn3(x0)

    o1, o2, o4 = _tc_head(h1, h2, h4, batch.reshape(1, N), Wl,
                          bl.reshape(1, H), Wc, bc.reshape(1, 10))
    return (o1, o2, o1, o4)
